# Initial kernel scaffold; baseline (speedup 1.0000x reference)
import jax
import jax.numpy as jnp
from jax.experimental import pallas as pl


def kernel(x, compress_neurons, router_W):
    raise NotImplementedError("write your pallas kernel here")


# speedup vs baseline: 5.8335x; 5.8335x over previous
import functools

import jax
import jax.numpy as jnp
from jax import lax
from jax.experimental import pallas as pl

D_MODEL = 768
RANK = 32
N_COMPRESS = 64
TOP_K = 2
S_TOKENS = 2048
BLK = 256


def _compressor_body(x_ref, rt_ref, wf_ref, scores_ref, w_ref, idx_ref, out_ref):
    xb = x_ref[...]
    scores = jnp.dot(xb, rt_ref[...], preferred_element_type=jnp.float32)
    scores_ref[...] = scores

    col = lax.broadcasted_iota(jnp.int32, scores.shape, 1)
    m1 = jnp.max(scores, axis=1, keepdims=True)
    i1 = jnp.min(jnp.where(scores == m1, col, N_COMPRESS), axis=1, keepdims=True)
    masked = jnp.where(col == i1, -jnp.inf, scores)
    m2 = jnp.max(masked, axis=1, keepdims=True)
    i2 = jnp.min(jnp.where(masked == m2, col, N_COMPRESS), axis=1, keepdims=True)

    e = jnp.exp(m2 - m1)
    w1 = 1.0 / (1.0 + e)
    w2 = e * w1
    w_ref[...] = jnp.concatenate([w1, w2], axis=1)
    idx_ref[...] = jnp.concatenate([i1, i2], axis=1)

    proj = jnp.dot(xb, wf_ref[...], preferred_element_type=jnp.float32)
    colp = lax.broadcasted_iota(jnp.int32, proj.shape, 1)
    ce = colp // RANK
    wexp = jnp.where(ce == i1, w1, 0.0) + jnp.where(ce == i2, w2, 0.0)
    weighted = proj * wexp

    rc = lax.broadcasted_iota(jnp.int32, (N_COMPRESS * RANK, RANK), 0) % RANK
    rr = lax.broadcasted_iota(jnp.int32, (N_COMPRESS * RANK, RANK), 1)
    sel = (rc == rr).astype(jnp.float32)
    out_ref[...] = jnp.dot(weighted, sel, preferred_element_type=jnp.float32)


@functools.partial(jax.jit, static_argnames=("interpret",))
def kernel(x, compress_neurons, router_W, interpret=False):
    b, s, d = x.shape
    xs = x.reshape(s, d)
    rt = router_W.T
    wf = compress_neurons.transpose(1, 0, 2).reshape(d, N_COMPRESS * RANK)

    grid = (s // BLK,)
    scores, weights, idx, out = pl.pallas_call(
        _compressor_body,
        grid=grid,
        in_specs=[
            pl.BlockSpec((BLK, d), lambda i: (i, 0)),
            pl.BlockSpec((d, N_COMPRESS), lambda i: (0, 0)),
            pl.BlockSpec((d, N_COMPRESS * RANK), lambda i: (0, 0)),
        ],
        out_specs=[
            pl.BlockSpec((BLK, N_COMPRESS), lambda i: (i, 0)),
            pl.BlockSpec((BLK, TOP_K), lambda i: (i, 0)),
            pl.BlockSpec((BLK, TOP_K), lambda i: (i, 0)),
            pl.BlockSpec((BLK, RANK), lambda i: (i, 0)),
        ],
        out_shape=[
            jax.ShapeDtypeStruct((s, N_COMPRESS), jnp.float32),
            jax.ShapeDtypeStruct((s, TOP_K), jnp.float32),
            jax.ShapeDtypeStruct((s, TOP_K), jnp.int32),
            jax.ShapeDtypeStruct((s, RANK), jnp.float32),
        ],
        interpret=interpret,
    )(xs, rt, wf)

    return (out.reshape(b, s, RANK),
            weights.reshape(b, s, TOP_K),
            idx.reshape(b, s, TOP_K),
            scores.reshape(b, s, N_COMPRESS))
